# Initial kernel scaffold; baseline (speedup 1.0000x reference)
import functools

import jax
import jax.numpy as jnp
from jax import lax
from jax.experimental import pallas as pl
from jax.experimental.pallas import tpu as pltpu
from jax.experimental.pallas import tpu_sc as plsc

_N = 10000
_E = 320000
_DIN = 128
_DOUT = 256
_W = 144
_NROWS = 10016
_NW = 32
_CHUNK = 128
_CH = 79
_EPW = _CH * _CHUNK
_RPT = _NROWS // 16


def _make_sc_aggregate():
    mesh = plsc.VectorSubcoreMesh(core_axis_name="c", subcore_axis_name="s")

    @functools.partial(
        pl.kernel,
        mesh=mesh,
        out_type=jax.ShapeDtypeStruct((2, _NROWS, _W), jnp.float32),
        scratch_types=[
            pltpu.VMEM((_CH, _CHUNK), jnp.int32),
            pltpu.VMEM((_CH, _CHUNK), jnp.int32),
            pltpu.VMEM((_CHUNK, _W), jnp.float32),
            pltpu.VMEM_SHARED((_NROWS, _W), jnp.float32),
            pltpu.SemaphoreType.DMA,
        ],
    )
    def sc_body(hx_hbm, src_hbm, dst_hbm, out_hbm, src_v, dst_v, rows_v, acc, sem):
        c = lax.axis_index("c")
        s = lax.axis_index("s")
        w = c * 16 + s

        pltpu.sync_copy(src_hbm.at[w], src_v)
        pltpu.sync_copy(dst_hbm.at[w], dst_v)

        def zrow(i, carry):
            for k in range(_W // 16):
                rows_v[i, pl.ds(k * 16, 16)] = jnp.zeros((16,), jnp.float32)
            return carry

        lax.fori_loop(0, _CHUNK, zrow, 0)

        row0 = s * _RPT
        off = 0
        while off < _RPT:
            n = min(_CHUNK, _RPT - off)
            pltpu.sync_copy(rows_v.at[pl.ds(0, n)], acc.at[pl.ds(row0 + off, n)])
            off += n

        plsc.subcore_barrier()

        def chunk(j, carry):
            pltpu.async_copy(hx_hbm.at[src_v.at[j]], rows_v, sem).wait()
            pltpu.sync_copy(rows_v, acc.at[dst_v.at[j]], add=True)
            return carry

        lax.fori_loop(0, _CH, chunk, 0)

        plsc.subcore_barrier()

        pltpu.sync_copy(acc.at[pl.ds(row0, _RPT)], out_hbm.at[c, pl.ds(row0, _RPT)])

    return sc_body


_sc_aggregate = _make_sc_aggregate()


def _finish_body(p0_ref, p1_ref, h_ref, g_ref, b_ref, o_ref):
    ssum = p0_ref[0] + p1_ref[0]
    msg = ssum[:, :_DIN]
    deg = ssum[:, _DIN:_DIN + 1]
    safe = jnp.maximum(deg, 1.0)
    inv2 = jnp.where(deg > 0, 1.0 / (safe * safe), 0.0)
    ahn = msg * inv2
    hb = h_ref[...]
    hc = jnp.concatenate([hb, ahn], axis=1)
    mu = jnp.mean(hc, axis=1, keepdims=True)
    d = hc - mu
    var = jnp.mean(d * d, axis=1, keepdims=True)
    o_ref[...] = d * lax.rsqrt(var + 1e-5) * g_ref[...] + b_ref[...]


def _finish(partials, h, gamma2, beta2):
    R = 1000
    grid = (_N // R,)
    return pl.pallas_call(
        _finish_body,
        grid=grid,
        in_specs=[
            pl.BlockSpec((1, R, _W), lambda i: (0, i, 0)),
            pl.BlockSpec((1, R, _W), lambda i: (1, i, 0)),
            pl.BlockSpec((R, _DIN), lambda i: (i, 0)),
            pl.BlockSpec((1, _DOUT), lambda i: (0, 0)),
            pl.BlockSpec((1, _DOUT), lambda i: (0, 0)),
        ],
        out_specs=pl.BlockSpec((R, _DOUT), lambda i: (i, 0)),
        out_shape=jax.ShapeDtypeStruct((_N, _DOUT), jnp.float32),
    )(partials, partials, h, gamma2, beta2)


def kernel(h, edge_index, ln_gamma, ln_beta):
    src = edge_index[0]
    dst = edge_index[1]

    hx = jnp.concatenate(
        [h, jnp.ones((_N, 1), jnp.float32), jnp.zeros((_N, _W - _DIN - 1), jnp.float32)],
        axis=1,
    )
    hx = jnp.concatenate([hx, jnp.zeros((_NROWS - _N, _W), jnp.float32)], axis=0)

    pad = _NW * _EPW - _E
    srcp = jnp.concatenate([src, jnp.full((pad,), _N, jnp.int32)]).reshape(_NW, _CH, _CHUNK)
    dstp = jnp.concatenate([dst, jnp.full((pad,), _N, jnp.int32)]).reshape(_NW, _CH, _CHUNK)

    partials = _sc_aggregate(hx, srcp, dstp)
    return _finish(partials, h, ln_gamma.reshape(1, _DOUT), ln_beta.reshape(1, _DOUT))


# speedup vs baseline: 4.9556x; 4.9556x over previous
import functools

import jax
import jax.numpy as jnp
from jax import lax
from jax.experimental import pallas as pl
from jax.experimental.pallas import tpu as pltpu
from jax.experimental.pallas import tpu_sc as plsc

_N = 10000
_E = 320000
_DIN = 128
_DOUT = 256
_W = 144
_NROWS = 10240
_NW = 32
_CHUNK = 128
_CH = 79
_EPW = _CH * _CHUNK
_RPT = _NROWS // 16


def _make_sc_aggregate():
    mesh = plsc.VectorSubcoreMesh(core_axis_name="c", subcore_axis_name="s")

    @functools.partial(
        pl.kernel,
        mesh=mesh,
        out_type=jax.ShapeDtypeStruct((2, _NROWS, _W), jnp.float32),
        scratch_types=[
            pltpu.VMEM((_CH, _CHUNK), jnp.int32),
            pltpu.VMEM((_CH, _CHUNK), jnp.int32),
            pltpu.VMEM((_CHUNK, _W), jnp.float32),
            pltpu.VMEM_SHARED((_NROWS, _W), jnp.float32),
            pltpu.SemaphoreType.DMA,
        ],
        compiler_params=pltpu.CompilerParams(use_tc_tiling_on_sc=False),
    )
    def sc_body(hx_hbm, src_hbm, dst_hbm, out_hbm, src_v, dst_v, rows_v, acc, sem):
        c = lax.axis_index("c")
        s = lax.axis_index("s")
        w = c * 16 + s

        pltpu.sync_copy(src_hbm.at[w], src_v)
        pltpu.sync_copy(dst_hbm.at[w], dst_v)

        def zrow(i, carry):
            for k in range(_W // 16):
                rows_v[i, pl.ds(k * 16, 16)] = jnp.zeros((16,), jnp.float32)
            return carry

        lax.fori_loop(0, _CHUNK, zrow, 0)

        row0 = s * _RPT
        off = 0
        while off < _RPT:
            n = min(_CHUNK, _RPT - off)
            pltpu.sync_copy(rows_v.at[pl.ds(0, n)], acc.at[pl.ds(row0 + off, n)])
            off += n

        plsc.subcore_barrier()

        def chunk(j, carry):
            pltpu.async_copy(hx_hbm.at[src_v.at[j]], rows_v, sem).wait()
            pltpu.sync_copy(rows_v, acc.at[dst_v.at[j]], add=True)
            return carry

        lax.fori_loop(0, _CH, chunk, 0)

        plsc.subcore_barrier()

        pltpu.sync_copy(acc.at[pl.ds(row0, _RPT)], out_hbm.at[c, pl.ds(row0, _RPT)])

    return sc_body


_sc_aggregate = _make_sc_aggregate()


def _finish_body(p0_ref, p1_ref, h_ref, g_ref, b_ref, o_ref):
    ssum = p0_ref[0] + p1_ref[0]
    msg = ssum[:, :_DIN]
    deg = ssum[:, _DIN:_DIN + 1]
    safe = jnp.maximum(deg, 1.0)
    inv2 = jnp.where(deg > 0, 1.0 / (safe * safe), 0.0)
    ahn = msg * inv2
    hb = h_ref[...]
    hc = jnp.concatenate([hb, ahn], axis=1)
    mu = jnp.mean(hc, axis=1, keepdims=True)
    d = hc - mu
    var = jnp.mean(d * d, axis=1, keepdims=True)
    o_ref[...] = d * lax.rsqrt(var + 1e-5) * g_ref[...] + b_ref[...]


def _finish(partials, h, gamma2, beta2):
    R = 1000
    grid = (_N // R,)
    return pl.pallas_call(
        _finish_body,
        grid=grid,
        in_specs=[
            pl.BlockSpec((1, R, _W), lambda i: (0, i, 0)),
            pl.BlockSpec((1, R, _W), lambda i: (1, i, 0)),
            pl.BlockSpec((R, _DIN), lambda i: (i, 0)),
            pl.BlockSpec((1, _DOUT), lambda i: (0, 0)),
            pl.BlockSpec((1, _DOUT), lambda i: (0, 0)),
        ],
        out_specs=pl.BlockSpec((R, _DOUT), lambda i: (i, 0)),
        out_shape=jax.ShapeDtypeStruct((_N, _DOUT), jnp.float32),
    )(partials, partials, h, gamma2, beta2)


def kernel(h, edge_index, ln_gamma, ln_beta):
    src = edge_index[0]
    dst = edge_index[1]

    hx = jnp.concatenate(
        [h, jnp.ones((_N, 1), jnp.float32), jnp.zeros((_N, _W - _DIN - 1), jnp.float32)],
        axis=1,
    )
    hx = jnp.concatenate([hx, jnp.zeros((_NROWS - _N, _W), jnp.float32)], axis=0)

    pad = _NW * _EPW - _E
    srcp = jnp.concatenate([src, jnp.full((pad,), _N, jnp.int32)]).reshape(_NW, _CH, _CHUNK)
    dstp = jnp.concatenate([dst, jnp.full((pad,), _N, jnp.int32)]).reshape(_NW, _CH, _CHUNK)

    partials = _sc_aggregate(hx, srcp, dstp)
    return _finish(partials, h, ln_gamma.reshape(1, _DOUT), ln_beta.reshape(1, _DOUT))


# speedup vs baseline: 4.9612x; 1.0011x over previous
import functools

import jax
import jax.numpy as jnp
from jax import lax
from jax.experimental import pallas as pl
from jax.experimental.pallas import tpu as pltpu
from jax.experimental.pallas import tpu_sc as plsc

_N = 10000
_E = 320000
_DIN = 128
_DOUT = 256
_W = 144
_NROWS = 10112
_NW = 32
_CHUNK = 64
_CH = 158
_CHA = 160
_EPW = _CHA * _CHUNK
_RPT = _NROWS // 16


def _make_sc_aggregate():
    mesh = plsc.VectorSubcoreMesh(core_axis_name="c", subcore_axis_name="s")

    @functools.partial(
        pl.kernel,
        mesh=mesh,
        out_type=jax.ShapeDtypeStruct((2, _NROWS, _W), jnp.float32),
        scratch_types=[
            pltpu.VMEM((_CHA, _CHUNK), jnp.int32),
            pltpu.VMEM((_CHA, _CHUNK), jnp.int32),
            pltpu.VMEM((2 * _CHUNK, _W), jnp.float32),
            pltpu.VMEM_SHARED((_NROWS, _W), jnp.float32),
            pltpu.SemaphoreType.DMA,
            pltpu.SemaphoreType.DMA,
            pltpu.SemaphoreType.DMA,
        ],
        compiler_params=pltpu.CompilerParams(use_tc_tiling_on_sc=False),
    )
    def sc_body(hx_hbm, src_hbm, dst_hbm, out_hbm,
                src_v, dst_v, rows_v, acc, sem0, sem1, semi):
        buf0 = rows_v.at[pl.ds(0, _CHUNK)]
        buf1 = rows_v.at[pl.ds(_CHUNK, _CHUNK)]
        c = lax.axis_index("c")
        s = lax.axis_index("s")
        w = c * 16 + s

        pltpu.async_copy(src_hbm.at[w], src_v, semi)
        pltpu.async_copy(dst_hbm.at[w], dst_v, semi)

        def zrow(i, carry):
            for k in range(_W // 16):
                rows_v[i, pl.ds(k * 16, 16)] = jnp.zeros((16,), jnp.float32)
            return carry

        lax.fori_loop(0, 2 * _CHUNK, zrow, 0)

        row0 = s * _RPT
        off = 0
        while off < _RPT:
            n = min(2 * _CHUNK, _RPT - off)
            pltpu.sync_copy(rows_v.at[pl.ds(0, n)], acc.at[pl.ds(row0 + off, n)])
            off += n

        pltpu.make_async_copy(src_hbm.at[w], src_v, semi).wait()
        pltpu.make_async_copy(dst_hbm.at[w], dst_v, semi).wait()

        plsc.subcore_barrier()

        def gather(j, buf, sem):
            pltpu.async_copy(hx_hbm.at[src_v.at[j]], buf, sem)

        def drain(buf, sem):
            pltpu.make_async_copy(hx_hbm.at[src_v.at[0]], buf, sem).wait()

        gather(0, buf0, sem0)

        def pair(jj, carry):
            j = jj * 2
            gather(j + 1, buf1, sem1)
            drain(buf0, sem0)
            pltpu.sync_copy(buf0, acc.at[dst_v.at[j]], add=True)
            gather(j + 2, buf0, sem0)
            drain(buf1, sem1)
            pltpu.sync_copy(buf1, acc.at[dst_v.at[j + 1]], add=True)
            return carry

        lax.fori_loop(0, _CH // 2, pair, 0)
        drain(buf0, sem0)

        plsc.subcore_barrier()

        pltpu.sync_copy(acc.at[pl.ds(row0, _RPT)], out_hbm.at[c, pl.ds(row0, _RPT)])

    return sc_body


_sc_aggregate = _make_sc_aggregate()


def _finish_body(p0_ref, p1_ref, h_ref, g_ref, b_ref, o_ref):
    ssum = p0_ref[0] + p1_ref[0]
    msg = ssum[:, :_DIN]
    deg = ssum[:, _DIN:_DIN + 1]
    safe = jnp.maximum(deg, 1.0)
    inv2 = jnp.where(deg > 0, 1.0 / (safe * safe), 0.0)
    ahn = msg * inv2
    hb = h_ref[...]
    hc = jnp.concatenate([hb, ahn], axis=1)
    mu = jnp.mean(hc, axis=1, keepdims=True)
    d = hc - mu
    var = jnp.mean(d * d, axis=1, keepdims=True)
    o_ref[...] = d * lax.rsqrt(var + 1e-5) * g_ref[...] + b_ref[...]


def _finish(partials, h, gamma2, beta2):
    R = 1000
    grid = (_N // R,)
    return pl.pallas_call(
        _finish_body,
        grid=grid,
        in_specs=[
            pl.BlockSpec((1, R, _W), lambda i: (0, i, 0)),
            pl.BlockSpec((1, R, _W), lambda i: (1, i, 0)),
            pl.BlockSpec((R, _DIN), lambda i: (i, 0)),
            pl.BlockSpec((1, _DOUT), lambda i: (0, 0)),
            pl.BlockSpec((1, _DOUT), lambda i: (0, 0)),
        ],
        out_specs=pl.BlockSpec((R, _DOUT), lambda i: (i, 0)),
        out_shape=jax.ShapeDtypeStruct((_N, _DOUT), jnp.float32),
    )(partials, partials, h, gamma2, beta2)


def kernel(h, edge_index, ln_gamma, ln_beta):
    src = edge_index[0]
    dst = edge_index[1]

    hx = jnp.concatenate(
        [h, jnp.ones((_N, 1), jnp.float32), jnp.zeros((_N, _W - _DIN - 1), jnp.float32)],
        axis=1,
    )
    hx = jnp.concatenate([hx, jnp.zeros((_NROWS - _N, _W), jnp.float32)], axis=0)

    epw_real = _E // _NW
    padw = _EPW - epw_real
    fill = jnp.full((_NW, padw), _N, jnp.int32)
    srcp = jnp.concatenate([src.reshape(_NW, epw_real), fill], axis=1).reshape(_NW, _CHA, _CHUNK)
    dstp = jnp.concatenate([dst.reshape(_NW, epw_real), fill], axis=1).reshape(_NW, _CHA, _CHUNK)

    partials = _sc_aggregate(hx, srcp, dstp)
    return _finish(partials, h, ln_gamma.reshape(1, _DOUT), ln_beta.reshape(1, _DOUT))


# speedup vs baseline: 8.6264x; 1.7388x over previous
import functools

import jax
import jax.numpy as jnp
from jax import lax
from jax.experimental import pallas as pl
from jax.experimental.pallas import tpu as pltpu
from jax.experimental.pallas import tpu_sc as plsc

_N = 10000
_E = 320000
_DIN = 128
_DOUT = 256
_HW = 64
_W = 80
_NROWS = 10112
_NT = 16
_CHUNK = 48
_BCH = 60
_NB = 7
_EPT = _NB * _BCH * _CHUNK
_RPT = _NROWS // _NT


def _make_sc_aggregate():
    mesh = plsc.VectorSubcoreMesh(core_axis_name="c", subcore_axis_name="s")

    @functools.partial(
        pl.kernel,
        mesh=mesh,
        out_type=jax.ShapeDtypeStruct((2, _NROWS, _W), jnp.float32),
        scratch_types=[
            pltpu.VMEM((_BCH, _CHUNK), jnp.int32),
            pltpu.VMEM((_BCH, _CHUNK), jnp.int32),
            pltpu.VMEM((_BCH, _CHUNK), jnp.int32),
            pltpu.VMEM((_BCH, _CHUNK), jnp.int32),
            pltpu.VMEM((2 * _CHUNK, _W), jnp.float32),
            pltpu.VMEM_SHARED((_NROWS, _W), jnp.float32),
            pltpu.VMEM_SHARED((_NROWS, _W), jnp.float32),
            pltpu.SemaphoreType.DMA,
            pltpu.SemaphoreType.DMA,
            pltpu.SemaphoreType.DMA,
        ],
        compiler_params=pltpu.CompilerParams(use_tc_tiling_on_sc=False),
    )
    def sc_body(hseg_hbm, src_hbm, dst_hbm, out_hbm,
                sidx_a, didx_a, sidx_b, didx_b, rows_v, h_spm, acc,
                sem0, sem1, semi):
        c = lax.axis_index("c")
        s = lax.axis_index("s")
        buf0 = rows_v.at[pl.ds(0, _CHUNK)]
        buf1 = rows_v.at[pl.ds(_CHUNK, _CHUNK)]
        row0 = s * _RPT

        pltpu.async_copy(hseg_hbm.at[c, pl.ds(row0, _RPT)],
                         h_spm.at[pl.ds(row0, _RPT)], semi)
        pltpu.async_copy(src_hbm.at[s, pl.ds(0, _BCH)], sidx_a, sem0)
        pltpu.async_copy(dst_hbm.at[s, pl.ds(0, _BCH)], didx_a, sem0)

        def zrow(i, carry):
            for k in range(_W // 16):
                rows_v[i, pl.ds(k * 16, 16)] = jnp.zeros((16,), jnp.float32)
            return carry

        lax.fori_loop(0, 2 * _CHUNK, zrow, 0)

        off = 0
        while off < _RPT:
            n = min(2 * _CHUNK, _RPT - off)
            pltpu.sync_copy(rows_v.at[pl.ds(0, n)], acc.at[pl.ds(row0 + off, n)])
            off += n

        pltpu.make_async_copy(hseg_hbm.at[c, pl.ds(row0, _RPT)],
                              h_spm.at[pl.ds(row0, _RPT)], semi).wait()
        plsc.subcore_barrier()

        def gather(sidx, j, buf, sem):
            pltpu.async_copy(h_spm.at[sidx.at[j]], buf, sem)

        def drain(buf, sem):
            pltpu.make_async_copy(h_spm.at[pl.ds(0, _CHUNK)], buf, sem).wait()

        def scat(didx, j, buf):
            pltpu.sync_copy(buf, acc.at[didx.at[j]], add=True)

        idx_bufs = [(sidx_a, didx_a), (sidx_b, didx_b)]
        for b in range(_NB):
            sidx, didx = idx_bufs[b % 2]
            sidx_n, didx_n = idx_bufs[(b + 1) % 2]
            pltpu.make_async_copy(src_hbm.at[s, pl.ds(0, _BCH)], sidx,
                                  sem0).wait()
            pltpu.make_async_copy(dst_hbm.at[s, pl.ds(0, _BCH)], didx,
                                  sem0).wait()
            if b + 1 < _NB:
                nb0 = (b + 1) * _BCH
                pltpu.async_copy(src_hbm.at[s, pl.ds(nb0, _BCH)], sidx_n, sem0)
                pltpu.async_copy(dst_hbm.at[s, pl.ds(nb0, _BCH)], didx_n, sem0)

            gather(sidx, 0, buf0, sem1)

            def pair(jj, carry):
                j = jj * 2
                gather(sidx, j + 1, buf1, semi)
                drain(buf0, sem1)
                scat(didx, j, buf0)
                gather(sidx, j + 2, buf0, sem1)
                drain(buf1, semi)
                scat(didx, j + 1, buf1)
                return carry

            lax.fori_loop(0, _BCH // 2 - 1, pair, 0)
            gather(sidx, _BCH - 1, buf1, semi)
            drain(buf0, sem1)
            scat(didx, _BCH - 2, buf0)
            drain(buf1, semi)
            scat(didx, _BCH - 1, buf1)

        plsc.subcore_barrier()

        pltpu.sync_copy(acc.at[pl.ds(row0, _RPT)], out_hbm.at[c, pl.ds(row0, _RPT)])

    return sc_body


_sc_aggregate = _make_sc_aggregate()


def _finish_body(p0_ref, p1_ref, h_ref, g_ref, b_ref, o_ref):
    p0 = p0_ref[0]
    p1 = p1_ref[0]
    msg = jnp.concatenate([p0[:, :_HW], p1[:, :_HW]], axis=1)
    deg = p0[:, _HW:_HW + 1]
    safe = jnp.maximum(deg, 1.0)
    inv2 = jnp.where(deg > 0, 1.0 / (safe * safe), 0.0)
    ahn = msg * inv2
    hb = h_ref[...]
    hc = jnp.concatenate([hb, ahn], axis=1)
    mu = jnp.mean(hc, axis=1, keepdims=True)
    d = hc - mu
    var = jnp.mean(d * d, axis=1, keepdims=True)
    o_ref[...] = d * lax.rsqrt(var + 1e-5) * g_ref[...] + b_ref[...]


def _finish(partials, h, gamma2, beta2):
    R = 1000
    grid = (_N // R,)
    return pl.pallas_call(
        _finish_body,
        grid=grid,
        in_specs=[
            pl.BlockSpec((1, R, _W), lambda i: (0, i, 0)),
            pl.BlockSpec((1, R, _W), lambda i: (1, i, 0)),
            pl.BlockSpec((R, _DIN), lambda i: (i, 0)),
            pl.BlockSpec((1, _DOUT), lambda i: (0, 0)),
            pl.BlockSpec((1, _DOUT), lambda i: (0, 0)),
        ],
        out_specs=pl.BlockSpec((R, _DOUT), lambda i: (i, 0)),
        out_shape=jax.ShapeDtypeStruct((_N, _DOUT), jnp.float32),
    )(partials, partials, h, gamma2, beta2)


def kernel(h, edge_index, ln_gamma, ln_beta):
    src = edge_index[0]
    dst = edge_index[1]

    ones = jnp.ones((_N, 1), jnp.float32)
    cpad = jnp.zeros((_N, _W - _HW - 1), jnp.float32)
    hseg = jnp.stack([
        jnp.concatenate([h[:, :_HW], ones, cpad], axis=1),
        jnp.concatenate([h[:, _HW:], ones, cpad], axis=1),
    ])
    hseg = jnp.concatenate(
        [hseg, jnp.zeros((2, _NROWS - _N, _W), jnp.float32)], axis=1)

    ept_real = _E // _NT
    fill = jnp.full((_NT, _EPT - ept_real), _N, jnp.int32)
    srcp = jnp.concatenate([src.reshape(_NT, ept_real), fill],
                           axis=1).reshape(_NT, _NB * _BCH, _CHUNK)
    dstp = jnp.concatenate([dst.reshape(_NT, ept_real), fill],
                           axis=1).reshape(_NT, _NB * _BCH, _CHUNK)

    partials = _sc_aggregate(hseg, srcp, dstp)
    return _finish(partials, h, ln_gamma.reshape(1, _DOUT), ln_beta.reshape(1, _DOUT))


# speedup vs baseline: 9.9200x; 1.1500x over previous
import functools

import jax
import jax.numpy as jnp
from jax import lax
from jax.experimental import pallas as pl
from jax.experimental.pallas import tpu as pltpu
from jax.experimental.pallas import tpu_sc as plsc

_N = 10000
_E = 320000
_DIN = 128
_DOUT = 256
_HW = 64
_NROWS = 10112
_NT = 16
_CHUNK = 48
_BCH = 60
_NB = 7
_EPT = _NB * _BCH * _CHUNK
_RPT = _NROWS // _NT


def _make_sc_aggregate():
    mesh = plsc.VectorSubcoreMesh(core_axis_name="c", subcore_axis_name="s")

    @functools.partial(
        pl.kernel,
        mesh=mesh,
        out_type=(
            jax.ShapeDtypeStruct((2, _NROWS, _HW), jnp.float32),
            jax.ShapeDtypeStruct((2, _NROWS), jnp.float32),
        ),
        scratch_types=[
            pltpu.VMEM((_BCH, _CHUNK), jnp.int32),
            pltpu.VMEM((_BCH, _CHUNK), jnp.int32),
            pltpu.VMEM((_BCH, _CHUNK), jnp.int32),
            pltpu.VMEM((_BCH, _CHUNK), jnp.int32),
            pltpu.VMEM((2 * _CHUNK, _HW), jnp.float32),
            pltpu.VMEM((_CHUNK,), jnp.float32),
            pltpu.VMEM_SHARED((_NROWS, _HW), jnp.float32),
            pltpu.VMEM_SHARED((_NROWS, _HW), jnp.float32),
            pltpu.VMEM_SHARED((_NROWS,), jnp.float32),
            pltpu.SemaphoreType.DMA,
            pltpu.SemaphoreType.DMA,
            pltpu.SemaphoreType.DMA,
            pltpu.SemaphoreType.DMA,
        ],
        compiler_params=pltpu.CompilerParams(use_tc_tiling_on_sc=False),
    )
    def sc_body(hp_hbm, src_hbm, dst_hbm, out_hbm, outd_hbm,
                sidx_a, didx_a, sidx_b, didx_b, rows_v, ones_v,
                h_spm, acc, accd, sem0, sem1, semi, semd):
        c = lax.axis_index("c")
        s = lax.axis_index("s")
        buf0 = rows_v.at[pl.ds(0, _CHUNK)]
        buf1 = rows_v.at[pl.ds(_CHUNK, _CHUNK)]
        row0 = s * _RPT

        pltpu.async_copy(hp_hbm.at[pl.ds(row0, _RPT), pl.ds(c * _HW, _HW)],
                         h_spm.at[pl.ds(row0, _RPT)], semi)
        pltpu.async_copy(src_hbm.at[s, pl.ds(0, _BCH)], sidx_a, sem0)
        pltpu.async_copy(dst_hbm.at[s, pl.ds(0, _BCH)], didx_a, sem0)

        def zrow(i, carry):
            for k in range(_HW // 16):
                rows_v[i, pl.ds(k * 16, 16)] = jnp.zeros((16,), jnp.float32)
            return carry

        lax.fori_loop(0, 2 * _CHUNK, zrow, 0)
        for k in range(_CHUNK // 16):
            ones_v[pl.ds(k * 16, 16)] = jnp.ones((16,), jnp.float32)

        off = 0
        while off < _RPT:
            n = min(2 * _CHUNK, _RPT - off)
            pltpu.sync_copy(rows_v.at[pl.ds(0, n)], acc.at[pl.ds(row0 + off, n)])
            off += n
        zrow64 = rows_v.at[0]
        for doff in (0, 64, 128, 192, 256, 320, 384, 448, 512, 568):
            pltpu.sync_copy(zrow64, accd.at[pl.ds(row0 + doff, 64)])

        pltpu.make_async_copy(hp_hbm.at[pl.ds(row0, _RPT), pl.ds(0, _HW)],
                              h_spm.at[pl.ds(row0, _RPT)], semi).wait()
        plsc.subcore_barrier()

        def gather(sidx, j, buf, sem):
            pltpu.async_copy(h_spm.at[sidx.at[j]], buf, sem)

        def drain(buf, sem):
            pltpu.make_async_copy(h_spm.at[pl.ds(0, _CHUNK)], buf, sem).wait()

        def scat(didx, j, buf):
            pltpu.async_copy(ones_v, accd.at[didx.at[j]], semd, add=True)
            pltpu.sync_copy(buf, acc.at[didx.at[j]], add=True)

        idx_bufs = [(sidx_a, didx_a), (sidx_b, didx_b)]
        for b in range(_NB):
            sidx, didx = idx_bufs[b % 2]
            sidx_n, didx_n = idx_bufs[(b + 1) % 2]
            pltpu.make_async_copy(src_hbm.at[s, pl.ds(0, _BCH)], sidx,
                                  sem0).wait()
            pltpu.make_async_copy(dst_hbm.at[s, pl.ds(0, _BCH)], didx,
                                  sem0).wait()
            if b + 1 < _NB:
                nb0 = (b + 1) * _BCH
                pltpu.async_copy(src_hbm.at[s, pl.ds(nb0, _BCH)], sidx_n, sem0)
                pltpu.async_copy(dst_hbm.at[s, pl.ds(nb0, _BCH)], didx_n, sem0)

            gather(sidx, 0, buf0, sem1)

            def pair(jj, carry):
                j = jj * 2
                gather(sidx, j + 1, buf1, semi)
                drain(buf0, sem1)
                scat(didx, j, buf0)
                gather(sidx, j + 2, buf0, sem1)
                drain(buf1, semi)
                scat(didx, j + 1, buf1)
                return carry

            lax.fori_loop(0, _BCH // 2 - 1, pair, 0)
            gather(sidx, _BCH - 1, buf1, semi)
            drain(buf0, sem1)
            scat(didx, _BCH - 2, buf0)
            drain(buf1, semi)
            scat(didx, _BCH - 1, buf1)

            def ddrain(i, carry):
                pltpu.make_async_copy(ones_v, accd.at[pl.ds(0, _CHUNK)],
                                      semd).wait()
                return carry

            lax.fori_loop(0, _BCH, ddrain, 0)

        plsc.subcore_barrier()

        pltpu.sync_copy(acc.at[pl.ds(row0, _RPT)], out_hbm.at[c, pl.ds(row0, _RPT)])
        pltpu.sync_copy(accd.at[pl.ds(row0, _RPT)], outd_hbm.at[c, pl.ds(row0, _RPT)])

    return sc_body


_sc_aggregate = _make_sc_aggregate()


def _finish_body(p0_ref, p1_ref, deg_ref, h_ref, g_ref, b_ref, o_ref):
    msg = jnp.concatenate([p0_ref[0], p1_ref[0]], axis=1)
    deg = deg_ref[0, 0][:, None]
    safe = jnp.maximum(deg, 1.0)
    inv2 = jnp.where(deg > 0, 1.0 / (safe * safe), 0.0)
    ahn = msg * inv2
    hb = h_ref[...]
    hc = jnp.concatenate([hb, ahn], axis=1)
    mu = jnp.mean(hc, axis=1, keepdims=True)
    d = hc - mu
    var = jnp.mean(d * d, axis=1, keepdims=True)
    o_ref[...] = d * lax.rsqrt(var + 1e-5) * g_ref[...] + b_ref[...]


def _finish(partials, deg3, h, gamma2, beta2):
    R = 1000
    grid = (_N // R,)
    return pl.pallas_call(
        _finish_body,
        grid=grid,
        in_specs=[
            pl.BlockSpec((1, R, _HW), lambda i: (0, i, 0)),
            pl.BlockSpec((1, R, _HW), lambda i: (1, i, 0)),
            pl.BlockSpec((1, 1, R), lambda i: (i, 0, 0)),
            pl.BlockSpec((R, _DIN), lambda i: (i, 0)),
            pl.BlockSpec((1, _DOUT), lambda i: (0, 0)),
            pl.BlockSpec((1, _DOUT), lambda i: (0, 0)),
        ],
        out_specs=pl.BlockSpec((R, _DOUT), lambda i: (i, 0)),
        out_shape=jax.ShapeDtypeStruct((_N, _DOUT), jnp.float32),
    )(partials, partials, deg3, h, gamma2, beta2)


def kernel(h, edge_index, ln_gamma, ln_beta):
    src = edge_index[0]
    dst = edge_index[1]

    hp = jnp.concatenate([h, jnp.zeros((_NROWS - _N, _DIN), jnp.float32)], axis=0)

    ept_real = _E // _NT
    fill = jnp.full((_NT, _EPT - ept_real), _N, jnp.int32)
    srcp = jnp.concatenate([src.reshape(_NT, ept_real), fill],
                           axis=1).reshape(_NT, _NB * _BCH, _CHUNK)
    dstp = jnp.concatenate([dst.reshape(_NT, ept_real), fill],
                           axis=1).reshape(_NT, _NB * _BCH, _CHUNK)

    partials, pdeg = _sc_aggregate(hp, srcp, dstp)
    deg3 = pdeg[0][: _N].reshape(_N // 1000, 1, 1000)
    return _finish(partials, deg3, h,
                   ln_gamma.reshape(1, _DOUT), ln_beta.reshape(1, _DOUT))
